# baseline (speedup 1.0000x reference)
import functools

import jax
import jax.numpy as jnp
from jax import lax
from jax.experimental import pallas as pl
from jax.experimental.pallas import tpu as pltpu

_DOT_LAST_LAST = (((1,), (1,)), ((), ()))

_VMEM_LIMIT = 60 * 1024 * 1024


def _round_up(v, m):
    return (v + m - 1) // m * m


def _body(x_ref, w_ref, m_ref, o_ref, wg_ref):
    @pl.when(pl.program_id(1) == 0)
    def _():
        wg_ref[...] = jnp.tanh(w_ref[...]) * jax.nn.sigmoid(m_ref[...])

    o_ref[...] = lax.dot_general(
        x_ref[...], wg_ref[...],
        dimension_numbers=_DOT_LAST_LAST,
        preferred_element_type=jnp.float32,
        precision=lax.Precision.DEFAULT,
    )


def _nac_fused(x, w_, m_, tm):
    B, K = x.shape
    N = w_.shape[0]
    Bp = _round_up(B, 2 * tm)
    if Bp != B:
        x = jnp.pad(x, ((0, Bp - B), (0, 0)))
    nb = Bp // tm
    nb_half = nb // 2

    wfull = pl.BlockSpec((N, K), lambda j, i: (0, 0))
    yp = pl.pallas_call(
        _body,
        out_shape=jax.ShapeDtypeStruct((Bp, N), jnp.float32),
        grid=(2, nb_half),
        in_specs=[
            pl.BlockSpec((tm, K), lambda j, i: (j * nb_half + i, 0)),
            wfull,
            wfull,
        ],
        out_specs=pl.BlockSpec((tm, N), lambda j, i: (j * nb_half + i, 0)),
        scratch_shapes=[pltpu.VMEM((N, K), jnp.float32)],
        compiler_params=pltpu.CompilerParams(
            dimension_semantics=("parallel", "arbitrary"),
            vmem_limit_bytes=_VMEM_LIMIT,
        ),
    )(x, w_, m_)
    return yp[:B] if Bp != B else yp


def kernel(x, w_, m_):
    assert x.ndim == 2 and w_.shape == m_.shape and x.shape[1] == w_.shape[1]
    B = x.shape[0]
    tm = 2048 if B % 4096 == 0 else max(8, _round_up((B + 1) // 2, 8))
    return _nac_fused(x, w_, m_, tm)
